# baseline (speedup 1.0000x reference)
import jax
import jax.numpy as jnp
from jax import lax
from jax.experimental import pallas as pl

_M_TILE = 2048
_BIG_F32 = 1e9


def _chamfer_body(x1_ref, x2x_ref, x2y_ref, x2z_ref, d1_ref, i1_ref,
                  d2_ref, i2_ref):
    j = pl.program_id(1)
    x1 = x1_ref[0]

    dx = x1[:, 0:1] - x2x_ref[0]
    dy = x1[:, 1:2] - x2y_ref[0]
    dz = x1[:, 2:3] - x2z_ref[0]
    d = dx * dx + dy * dy + dz * dz

    n = d.shape[0]
    rmin = jnp.min(d, axis=1, keepdims=True)
    col_ids = lax.broadcasted_iota(jnp.int32, (1, _M_TILE), 1).astype(jnp.float32)
    ridx_f = jnp.min(jnp.where(d == rmin, col_ids, _BIG_F32), axis=1,
                     keepdims=True)
    ridx = ridx_f.astype(jnp.int32) + j * _M_TILE

    @pl.when(j == 0)
    def _init():
        d1_ref[0] = rmin
        i1_ref[0] = ridx

    @pl.when(j > 0)
    def _merge():
        prev = d1_ref[0]
        upd = rmin < prev
        d1_ref[0] = jnp.where(upd, rmin, prev)
        i1_ref[0] = jnp.where(upd, ridx, i1_ref[0])

    cmin = jnp.min(d, axis=0, keepdims=True)
    row_ids = lax.broadcasted_iota(jnp.int32, (n, 1), 0).astype(jnp.float32)
    cidx_f = jnp.min(jnp.where(d == cmin, row_ids, _BIG_F32), axis=0,
                     keepdims=True)
    d2_ref[0] = cmin
    i2_ref[0] = cidx_f.astype(jnp.int32)


import functools
from jax.experimental.pallas import tpu as pltpu
from jax.experimental.pallas import tpu_sc as plsc

_NC = 2
_NS = 16
_NW = _NC * _NS


def _make_sc_chamfer(B, N, M):
    QPW = N // _NW
    mesh = plsc.VectorSubcoreMesh(core_axis_name="c", subcore_axis_name="s")

    @functools.partial(
        pl.kernel,
        out_type=[
            jax.ShapeDtypeStruct((B * N,), jnp.float32),
            jax.ShapeDtypeStruct((B * N,), jnp.int32),
            jax.ShapeDtypeStruct((B * M,), jnp.float32),
            jax.ShapeDtypeStruct((B * M,), jnp.int32),
        ],
        mesh=mesh,
        scratch_types=[
            pltpu.VMEM((M,), jnp.float32),
            pltpu.VMEM((M,), jnp.float32),
            pltpu.VMEM((M,), jnp.float32),
            pltpu.VMEM((QPW,), jnp.float32),
            pltpu.VMEM((QPW,), jnp.float32),
            pltpu.VMEM((QPW,), jnp.float32),
            pltpu.VMEM((QPW,), jnp.float32),
            pltpu.VMEM((QPW,), jnp.int32),
        ],
    )
    def k(x1x, x1y, x1z, x2x, x2y, x2z, d1, i1, d2, i2,
          cx, cy, cz, qx, qy, qz, od, oi):
        wid = lax.axis_index("s") * _NC + lax.axis_index("c")

        def one_direction(qsx, qsy, qsz, csx, csy, csz, dout, iout, nq, nc):
            def per_batch(b, _):
                pltpu.sync_copy(csx.at[pl.ds(b * nc, nc)], cx)
                pltpu.sync_copy(csy.at[pl.ds(b * nc, nc)], cy)
                pltpu.sync_copy(csz.at[pl.ds(b * nc, nc)], cz)
                qbase = b * nq + wid * QPW
                pltpu.sync_copy(qsx.at[pl.ds(qbase, QPW)], qx)
                pltpu.sync_copy(qsy.at[pl.ds(qbase, QPW)], qy)
                pltpu.sync_copy(qsz.at[pl.ds(qbase, QPW)], qz)

                def per_group(g, _):
                    qxv = qx[pl.ds(g * 16, 16)]
                    qyv = qy[pl.ds(g * 16, 16)]
                    qzv = qz[pl.ds(g * 16, 16)]

                    def per_chunk(jc, carry):
                        mv, mi = carry
                        ccx = cx[pl.ds(jc * 16, 16)]
                        ccy = cy[pl.ds(jc * 16, 16)]
                        ccz = cz[pl.ds(jc * 16, 16)]
                        jbase = jc * 16
                        vals = []
                        for t in range(16):
                            lane = jnp.full((16,), t, jnp.int32)
                            cxv = ccx.at[lane].get(mode="promise_in_bounds")
                            cyv = ccy.at[lane].get(mode="promise_in_bounds")
                            czv = ccz.at[lane].get(mode="promise_in_bounds")
                            dx = qxv - cxv
                            dy = qyv - cyv
                            dz = qzv - czv
                            dd = dx * dx + dy * dy + dz * dz
                            vals.append((dd, jnp.full((16,), t, jnp.int32)))
                        while len(vals) > 1:
                            nxt = []
                            for a in range(0, len(vals), 2):
                                (va, ia), (vb, ib) = vals[a], vals[a + 1]
                                m = vb < va
                                nxt.append((jnp.where(m, vb, va),
                                            jnp.where(m, ib, ia)))
                            vals = nxt
                        cv, ci = vals[0]
                        m = cv < mv
                        return (jnp.where(m, cv, mv),
                                jnp.where(m, jbase + ci, mi))

                    mv0 = jnp.full((16,), jnp.inf, jnp.float32)
                    mi0 = jnp.zeros((16,), jnp.int32)
                    mv, mi = lax.fori_loop(0, nc // 16, per_chunk, (mv0, mi0))
                    od[pl.ds(g * 16, 16)] = mv
                    oi[pl.ds(g * 16, 16)] = mi
                    return 0

                lax.fori_loop(0, QPW // 16, per_group, 0)
                pltpu.sync_copy(od, dout.at[pl.ds(qbase, QPW)])
                pltpu.sync_copy(oi, iout.at[pl.ds(qbase, QPW)])
                return 0

            lax.fori_loop(0, B, per_batch, 0)

        one_direction(x1x, x1y, x1z, x2x, x2y, x2z, d1, i1, N, M)
        one_direction(x2x, x2y, x2z, x1x, x1y, x1z, d2, i2, M, N)

    return k


def _kernel_sc(xyz1, xyz2):
    B, N, _ = xyz1.shape
    M = xyz2.shape[1]
    x1x = xyz1[:, :, 0].reshape(B * N)
    x1y = xyz1[:, :, 1].reshape(B * N)
    x1z = xyz1[:, :, 2].reshape(B * N)
    x2x = xyz2[:, :, 0].reshape(B * M)
    x2y = xyz2[:, :, 1].reshape(B * M)
    x2z = xyz2[:, :, 2].reshape(B * M)
    d1, i1, d2, i2 = _make_sc_chamfer(B, N, M)(x1x, x1y, x1z, x2x, x2y, x2z)
    return (d1.reshape(B, N), d2.reshape(B, M),
            i1.reshape(B, N), i2.reshape(B, M))


def kernel(xyz1, xyz2):
    return _kernel_sc(xyz1, xyz2)


def _kernel_tc(xyz1, xyz2):
    B, N, _ = xyz1.shape
    M = xyz2.shape[1]
    x2x = xyz2[:, :, 0].reshape(B, 1, M)
    x2y = xyz2[:, :, 1].reshape(B, 1, M)
    x2z = xyz2[:, :, 2].reshape(B, 1, M)
    n_tiles = M // _M_TILE

    grid = (B, n_tiles)
    row_spec = pl.BlockSpec((1, 1, _M_TILE), lambda b, j: (b, 0, j))
    d1, i1, d2, i2 = pl.pallas_call(
        _chamfer_body,
        grid=grid,
        in_specs=[
            pl.BlockSpec((1, N, 3), lambda b, j: (b, 0, 0)),
            row_spec, row_spec, row_spec,
        ],
        out_specs=[
            pl.BlockSpec((1, N, 1), lambda b, j: (b, 0, 0)),
            pl.BlockSpec((1, N, 1), lambda b, j: (b, 0, 0)),
            pl.BlockSpec((1, 1, _M_TILE), lambda b, j: (b, 0, j)),
            pl.BlockSpec((1, 1, _M_TILE), lambda b, j: (b, 0, j)),
        ],
        out_shape=[
            jax.ShapeDtypeStruct((B, N, 1), jnp.float32),
            jax.ShapeDtypeStruct((B, N, 1), jnp.int32),
            jax.ShapeDtypeStruct((B, 1, M), jnp.float32),
            jax.ShapeDtypeStruct((B, 1, M), jnp.int32),
        ],
    )(xyz1, x2x, x2y, x2z)

    dist1 = d1.reshape(B, N)
    idx1 = i1.reshape(B, N)
    dist2 = d2.reshape(B, M)
    idx2 = i2.reshape(B, M)
    return (dist1, dist2, idx1, idx2)


# speedup vs baseline: 4.0214x; 4.0214x over previous
import functools

import jax
import jax.numpy as jnp
from jax import lax
from jax.experimental import pallas as pl
from jax.experimental.pallas import tpu as pltpu
from jax.experimental.pallas import tpu_sc as plsc

_M_TILE = 2048
_BIG_F32 = 1e9

_NC = 2
_NS = 16
_NW = _NC * _NS

_S1 = 1024
_S2 = 512


def _chamfer_body(x1_ref, x2x_ref, x2y_ref, x2z_ref, d1_ref, i1_ref,
                  d2_ref, i2_ref):
    j = pl.program_id(1)
    nj = pl.num_programs(1)
    x1 = x1_ref[0]

    dx = x1[:, 0:1] - x2x_ref[0]
    dy = x1[:, 1:2] - x2y_ref[0]
    dz = x1[:, 2:3] - x2z_ref[0]
    d = dx * dx + dy * dy + dz * dz

    n = d.shape[0]
    n1 = n - _S1

    dr = d[:n1]
    rmin = jnp.min(dr, axis=1, keepdims=True)
    col_ids = lax.broadcasted_iota(jnp.int32, (1, _M_TILE), 1).astype(jnp.float32)
    ridx_f = jnp.min(jnp.where(dr == rmin, col_ids, _BIG_F32), axis=1,
                     keepdims=True)
    ridx = ridx_f.astype(jnp.int32) + j * _M_TILE

    @pl.when(j == 0)
    def _init():
        d1_ref[0] = rmin
        i1_ref[0] = ridx

    @pl.when(j > 0)
    def _merge():
        prev = d1_ref[0]
        upd = rmin < prev
        d1_ref[0] = jnp.where(upd, rmin, prev)
        i1_ref[0] = jnp.where(upd, ridx, i1_ref[0])

    row_ids = lax.broadcasted_iota(jnp.int32, (n, 1), 0).astype(jnp.float32)

    @pl.when(j < nj - 1)
    def _cols_full():
        cmin = jnp.min(d, axis=0, keepdims=True)
        cidx_f = jnp.min(jnp.where(d == cmin, row_ids, _BIG_F32), axis=0,
                         keepdims=True)
        d2_ref[0] = cmin
        i2_ref[0] = cidx_f.astype(jnp.int32)

    @pl.when(j == nj - 1)
    def _cols_tail():
        dc = d[:, :_M_TILE - _S2]
        cmin = jnp.min(dc, axis=0, keepdims=True)
        cidx_f = jnp.min(jnp.where(dc == cmin, row_ids, _BIG_F32), axis=0,
                         keepdims=True)
        d2_ref[0, 0:1, :_M_TILE - _S2] = cmin
        i2_ref[0, 0:1, :_M_TILE - _S2] = cidx_f.astype(jnp.int32)


def _kernel_tc(xyz1, xyz2):
    B, N, _ = xyz1.shape
    M = xyz2.shape[1]
    N1 = N - _S1
    x2x = xyz2[:, :, 0].reshape(B, 1, M)
    x2y = xyz2[:, :, 1].reshape(B, 1, M)
    x2z = xyz2[:, :, 2].reshape(B, 1, M)
    n_tiles = M // _M_TILE

    grid = (B, n_tiles)
    row_spec = pl.BlockSpec((1, 1, _M_TILE), lambda b, j: (b, 0, j))
    d1, i1, d2, i2 = pl.pallas_call(
        _chamfer_body,
        grid=grid,
        in_specs=[
            pl.BlockSpec((1, N, 3), lambda b, j: (b, 0, 0)),
            row_spec, row_spec, row_spec,
        ],
        out_specs=[
            pl.BlockSpec((1, N1, 1), lambda b, j: (b, 0, 0)),
            pl.BlockSpec((1, N1, 1), lambda b, j: (b, 0, 0)),
            pl.BlockSpec((1, 1, _M_TILE), lambda b, j: (b, 0, j)),
            pl.BlockSpec((1, 1, _M_TILE), lambda b, j: (b, 0, j)),
        ],
        out_shape=[
            jax.ShapeDtypeStruct((B, N1, 1), jnp.float32),
            jax.ShapeDtypeStruct((B, N1, 1), jnp.int32),
            jax.ShapeDtypeStruct((B, 1, M), jnp.float32),
            jax.ShapeDtypeStruct((B, 1, M), jnp.int32),
        ],
    )(xyz1, x2x, x2y, x2z)
    return d1, i1, d2, i2


def _make_sc_chamfer(B, N, M):
    mesh = plsc.VectorSubcoreMesh(core_axis_name="c", subcore_axis_name="s")
    qpw1 = _S1 // _NW
    qpw2 = _S2 // _NW

    @functools.partial(
        pl.kernel,
        out_type=[
            jax.ShapeDtypeStruct((B * _S1,), jnp.float32),
            jax.ShapeDtypeStruct((B * _S1,), jnp.int32),
            jax.ShapeDtypeStruct((B * _S2,), jnp.float32),
            jax.ShapeDtypeStruct((B * _S2,), jnp.int32),
        ],
        mesh=mesh,
        scratch_types=[
            pltpu.VMEM((M,), jnp.float32),
            pltpu.VMEM((M,), jnp.float32),
            pltpu.VMEM((M,), jnp.float32),
            pltpu.VMEM((qpw1,), jnp.float32),
            pltpu.VMEM((qpw1,), jnp.float32),
            pltpu.VMEM((qpw1,), jnp.float32),
            pltpu.VMEM((qpw1,), jnp.float32),
            pltpu.VMEM((qpw1,), jnp.int32),
        ],
    )
    def k(x1x, x1y, x1z, x2x, x2y, x2z, d1, i1, d2, i2,
          cx, cy, cz, qx, qy, qz, od, oi):
        wid = lax.axis_index("s") * _NC + lax.axis_index("c")

        def one_direction(qsx, qsy, qsz, csx, csy, csz, dout, iout,
                          nq, nc, s, qpw):
            def per_batch(b, _):
                pltpu.sync_copy(csx.at[pl.ds(b * nc, nc)], cx)
                pltpu.sync_copy(csy.at[pl.ds(b * nc, nc)], cy)
                pltpu.sync_copy(csz.at[pl.ds(b * nc, nc)], cz)
                qbase = b * nq + (nq - s) + wid * qpw
                obase = b * s + wid * qpw
                pltpu.sync_copy(qsx.at[pl.ds(qbase, qpw)], qx.at[pl.ds(0, qpw)])
                pltpu.sync_copy(qsy.at[pl.ds(qbase, qpw)], qy.at[pl.ds(0, qpw)])
                pltpu.sync_copy(qsz.at[pl.ds(qbase, qpw)], qz.at[pl.ds(0, qpw)])

                def per_group(g, _):
                    qxv = qx[pl.ds(g * 16, 16)]
                    qyv = qy[pl.ds(g * 16, 16)]
                    qzv = qz[pl.ds(g * 16, 16)]

                    def per_chunk(jc, carry):
                        mv, mi = carry
                        ccx = cx[pl.ds(jc * 16, 16)]
                        ccy = cy[pl.ds(jc * 16, 16)]
                        ccz = cz[pl.ds(jc * 16, 16)]
                        jbase = jc * 16
                        for t in range(16):
                            lane = jnp.full((16,), t, jnp.int32)
                            cxv = ccx.at[lane].get(mode="promise_in_bounds")
                            cyv = ccy.at[lane].get(mode="promise_in_bounds")
                            czv = ccz.at[lane].get(mode="promise_in_bounds")
                            dx = qxv - cxv
                            dy = qyv - cyv
                            dz = qzv - czv
                            dd = dx * dx + dy * dy + dz * dz
                            m = dd < mv
                            mv = jnp.where(m, dd, mv)
                            mi = jnp.where(m, jbase + t, mi)
                        return (mv, mi)

                    mv0 = jnp.full((16,), jnp.inf, jnp.float32)
                    mi0 = jnp.zeros((16,), jnp.int32)
                    mv, mi = lax.fori_loop(0, nc // 16, per_chunk, (mv0, mi0))
                    od[pl.ds(g * 16, 16)] = mv
                    oi[pl.ds(g * 16, 16)] = mi
                    return 0

                lax.fori_loop(0, qpw // 16, per_group, 0)
                pltpu.sync_copy(od.at[pl.ds(0, qpw)], dout.at[pl.ds(obase, qpw)])
                pltpu.sync_copy(oi.at[pl.ds(0, qpw)], iout.at[pl.ds(obase, qpw)])
                return 0

            lax.fori_loop(0, B, per_batch, 0)

        one_direction(x1x, x1y, x1z, x2x, x2y, x2z, d1, i1, N, M, _S1, qpw1)
        one_direction(x2x, x2y, x2z, x1x, x1y, x1z, d2, i2, M, N, _S2, qpw2)

    return k


def _kernel_sc(xyz1, xyz2):
    B, N, _ = xyz1.shape
    M = xyz2.shape[1]
    x1x = xyz1[:, :, 0].reshape(B * N)
    x1y = xyz1[:, :, 1].reshape(B * N)
    x1z = xyz1[:, :, 2].reshape(B * N)
    x2x = xyz2[:, :, 0].reshape(B * M)
    x2y = xyz2[:, :, 1].reshape(B * M)
    x2z = xyz2[:, :, 2].reshape(B * M)
    return _make_sc_chamfer(B, N, M)(x1x, x1y, x1z, x2x, x2y, x2z)


def kernel(xyz1, xyz2):
    B, N, _ = xyz1.shape
    M = xyz2.shape[1]
    s1, i1s, s2, i2s = _kernel_sc(xyz1, xyz2)
    d1, i1, d2, i2 = _kernel_tc(xyz1, xyz2)

    dist1 = jnp.concatenate([d1.reshape(B, N - _S1), s1.reshape(B, _S1)], axis=1)
    idx1 = jnp.concatenate([i1.reshape(B, N - _S1), i1s.reshape(B, _S1)], axis=1)
    dist2 = jnp.concatenate([d2.reshape(B, M)[:, :M - _S2], s2.reshape(B, _S2)],
                            axis=1)
    idx2 = jnp.concatenate([i2.reshape(B, M)[:, :M - _S2], i2s.reshape(B, _S2)],
                           axis=1)
    return (dist1, dist2, idx1, idx2)


# speedup vs baseline: 4.4731x; 1.1123x over previous
import jax
import jax.numpy as jnp
from jax import lax
from jax.experimental import pallas as pl

_M_TILE = 2048
_BIG_F32 = 1e9


def _chamfer_body(x1_ref, x2t_ref, d1_ref, i1_ref, d2_ref, i2_ref):
    j = pl.program_id(1)
    x1 = x1_ref[0]
    x2 = x2t_ref[0]

    dx = x1[:, 0:1] - x2[0:1, :]
    dy = x1[:, 1:2] - x2[1:2, :]
    dz = x1[:, 2:3] - x2[2:3, :]
    d = dx * dx + dy * dy + dz * dz

    n = d.shape[0]
    rmin = jnp.min(d, axis=1, keepdims=True)
    col_ids = lax.broadcasted_iota(jnp.int32, (1, _M_TILE), 1).astype(jnp.float32)
    ridx_f = jnp.min(jnp.where(d == rmin, col_ids, _BIG_F32), axis=1,
                     keepdims=True)
    ridx = ridx_f.astype(jnp.int32) + j * _M_TILE

    @pl.when(j == 0)
    def _init():
        d1_ref[0] = rmin
        i1_ref[0] = ridx

    @pl.when(j > 0)
    def _merge():
        prev = d1_ref[0]
        upd = rmin < prev
        d1_ref[0] = jnp.where(upd, rmin, prev)
        i1_ref[0] = jnp.where(upd, ridx, i1_ref[0])

    cmin = jnp.min(d, axis=0, keepdims=True)
    row_ids = lax.broadcasted_iota(jnp.int32, (n, 1), 0).astype(jnp.float32)
    cidx_f = jnp.min(jnp.where(d == cmin, row_ids, _BIG_F32), axis=0,
                     keepdims=True)
    d2_ref[0] = cmin
    i2_ref[0] = cidx_f.astype(jnp.int32)


def kernel(xyz1, xyz2):
    B, N, _ = xyz1.shape
    M = xyz2.shape[1]
    xyz2t = xyz2.transpose(0, 2, 1)
    n_tiles = M // _M_TILE

    grid = (B, n_tiles)
    d1, i1, d2, i2 = pl.pallas_call(
        _chamfer_body,
        grid=grid,
        in_specs=[
            pl.BlockSpec((1, N, 3), lambda b, j: (b, 0, 0)),
            pl.BlockSpec((1, 3, _M_TILE), lambda b, j: (b, 0, j)),
        ],
        out_specs=[
            pl.BlockSpec((1, N, 1), lambda b, j: (b, 0, 0)),
            pl.BlockSpec((1, N, 1), lambda b, j: (b, 0, 0)),
            pl.BlockSpec((1, 1, _M_TILE), lambda b, j: (b, 0, j)),
            pl.BlockSpec((1, 1, _M_TILE), lambda b, j: (b, 0, j)),
        ],
        out_shape=[
            jax.ShapeDtypeStruct((B, N, 1), jnp.float32),
            jax.ShapeDtypeStruct((B, N, 1), jnp.int32),
            jax.ShapeDtypeStruct((B, 1, M), jnp.float32),
            jax.ShapeDtypeStruct((B, 1, M), jnp.int32),
        ],
    )(xyz1, xyz2t)

    dist1 = d1.reshape(B, N)
    idx1 = i1.reshape(B, N)
    dist2 = d2.reshape(B, M)
    idx2 = i2.reshape(B, M)
    return (dist1, dist2, idx1, idx2)
